# Initial kernel scaffold; baseline (speedup 1.0000x reference)
import jax
import jax.numpy as jnp
from jax.experimental import pallas as pl


def kernel(features, labels):
    raise NotImplementedError("write your pallas kernel here")


# speedup vs baseline: 2.3014x; 2.3014x over previous
import jax
import jax.numpy as jnp
from jax import lax
from jax.experimental import pallas as pl
from jax.experimental.pallas import tpu as pltpu
from jax.experimental.pallas import tpu_sc as plsc

N = 16384
D = 512
C = 100
CP = 128
NC = 2
NS = 16
NW = NC * NS
RPW = N // NW
CH = 64
NCH = RPW // CH
KV = D // 16
BR = 512
NB = N // BR
NEG = -3.0e38


def _seg_sum_body(feat_hbm, lab_hbm, sums_out, cnt_out, fbuf, labv, acc, cacc):
    cid = lax.axis_index("c")
    sid = lax.axis_index("s")
    wid = sid * NC + cid
    base = wid * RPW

    zero16 = jnp.zeros((16,), jnp.float32)
    one16 = jnp.ones((16,), jnp.float32)

    def zrow(i, carry):
        r = i // KV
        k = i % KV
        acc[r, pl.ds(k * 16, 16)] = zero16
        return carry

    lax.fori_loop(0, CP * KV, zrow, 0)

    def crow(i, carry):
        cacc[i, :] = zero16
        return carry

    lax.fori_loop(0, CP, crow, 0)

    pltpu.sync_copy(lab_hbm.at[pl.ds(base, RPW)], labv)

    def chunk(c, carry):
        pltpu.sync_copy(feat_hbm.at[pl.ds(base + c * CH, CH)], fbuf)

        def group(g, carry2):
            lv = labv[pl.ds(c * CH + g * 16, 16)]
            for j in range(16):
                l = lv[j]
                row = g * 16 + j
                for k in range(KV):
                    plsc.addupdate(acc.at[l, pl.ds(k * 16, 16)],
                                   fbuf[row, pl.ds(k * 16, 16)])
                plsc.addupdate(cacc.at[l, :], one16)
            return carry2

        lax.fori_loop(0, CH // 16, group, 0)
        return carry

    lax.fori_loop(0, NCH, chunk, 0)

    pltpu.sync_copy(acc, sums_out.at[wid])
    pltpu.sync_copy(cacc, cnt_out.at[wid])


_seg_sum = pl.kernel(
    _seg_sum_body,
    out_type=[
        jax.ShapeDtypeStruct((NW, CP, D), jnp.float32),
        jax.ShapeDtypeStruct((NW, CP, 16), jnp.float32),
    ],
    mesh=plsc.VectorSubcoreMesh(core_axis_name="c", subcore_axis_name="s"),
    scratch_types=[
        pltpu.VMEM((CH, D), jnp.float32),
        pltpu.VMEM((RPW,), jnp.int32),
        pltpu.VMEM((CP, D), jnp.float32),
        pltpu.VMEM((CP, 16), jnp.float32),
    ],
)


def _tc_body(lab_ref, sums_ref, cnt_ref, f_ref, out_ref, centers, mx, mn):
    i = pl.program_id(0)

    @pl.when(i == 0)
    def _():
        s = jnp.sum(sums_ref[...], axis=0)
        cnts = jnp.sum(cnt_ref[...], axis=0)
        c0 = cnts[:, 0:1]
        centers[...] = s / jnp.maximum(c0, 1.0)

    f = f_ref[...]
    fnorm = jnp.sum(f * f, axis=1, keepdims=True)
    g = lax.dot_general(f, centers[...], (((1,), (1,)), ((), ())),
                        precision=lax.Precision.HIGHEST,
                        preferred_element_type=jnp.float32)
    m = fnorm - 2.0 * g
    lab = lab_ref[0]
    cls = lax.broadcasted_iota(jnp.int32, (BR, CP), 1)
    mask = lab == cls
    pmax = jnp.max(jnp.where(mask, m, NEG), axis=0, keepdims=True)
    pmin = jnp.min(jnp.where(mask, m, -NEG), axis=0, keepdims=True)
    prev_mx = jnp.where(i == 0, jnp.full_like(pmax, NEG), mx[...])
    prev_mn = jnp.where(i == 0, jnp.full_like(pmin, -NEG), mn[...])
    mx[...] = jnp.maximum(prev_mx, pmax)
    mn[...] = jnp.minimum(prev_mn, pmin)

    @pl.when(i == NB - 1)
    def _():
        present = mx[...] > NEG * 0.5
        per_class = jnp.where(present, mx[...] - mn[...], 0.0)
        npres = jnp.sum(present.astype(jnp.float32), axis=1, keepdims=True)
        out_ref[...] = jnp.sum(per_class, axis=1, keepdims=True) / npres


def _tc_call(lab3d, sums_part, cnt_part, features):
    return pl.pallas_call(
        _tc_body,
        grid=(NB,),
        in_specs=[
            pl.BlockSpec((1, BR, 1), lambda i: (i, 0, 0)),
            pl.BlockSpec((NW, CP, D), lambda i: (0, 0, 0)),
            pl.BlockSpec((NW, CP, 16), lambda i: (0, 0, 0)),
            pl.BlockSpec((BR, D), lambda i: (i, 0)),
        ],
        out_specs=pl.BlockSpec((1, 1), lambda i: (0, 0)),
        out_shape=jax.ShapeDtypeStruct((1, 1), jnp.float32),
        scratch_shapes=[
            pltpu.VMEM((CP, D), jnp.float32),
            pltpu.VMEM((1, CP), jnp.float32),
            pltpu.VMEM((1, CP), jnp.float32),
        ],
        compiler_params=pltpu.CompilerParams(
            dimension_semantics=("arbitrary",),
        ),
    )(lab3d, sums_part, cnt_part, features)


def kernel(features, labels):
    labels32 = labels.astype(jnp.int32)
    sums_part, cnt_part = _seg_sum(features, labels32)
    lab3d = labels32.reshape(NB, BR, 1)
    loss = _tc_call(lab3d, sums_part, cnt_part, features)
    return loss[0, 0]
